# baseline (speedup 1.0000x reference)
import functools
import numpy as np
import jax
import jax.numpy as jnp
from jax import lax
from jax.experimental import pallas as pl
from jax.experimental.pallas import tpu as pltpu
from jax.experimental.pallas import tpu_sc as plsc

B = 256
INPUT_SIZE = 4096
HIDDEN = 2048
OUTPUT = 1024
R_SZ = HIDDEN // 4
S_SZ = HIDDEN // 4
C_SZ = HIDDEN - R_SZ - S_SZ
BASE = 16
ENC = 256

_INTERPRET = False

_SPECS = [
    ("wr", ENC, R_SZ), ("rr", R_SZ, R_SZ), ("wc", R_SZ, C_SZ),
    ("rc", C_SZ, C_SZ), ("ws", C_SZ, S_SZ), ("rs", S_SZ, S_SZ),
    ("pr", R_SZ, R_SZ), ("pc", C_SZ, C_SZ), ("ps", S_SZ, S_SZ),
]
_SEG_ORDER = ["rc", "pc", "wc", "ws", "rr", "rs", "pr", "ps", "wr"]

_TASK_WORDS = 65536


def _seg_layout():
    by_name = {nm: (fi, fo) for nm, fi, fo in _SPECS}
    segs = []
    seg_off = 0
    for nm in _SEG_ORDER:
        fi, fo = by_name[nm]
        nnz = max(int(fi * fo * 0.01), 1)
        nnzp = -(-nnz // 16) * 16
        nrows = _TASK_WORDS // fi
        segs.append(dict(nm=nm, nnz=nnz, nnzp=nnzp, in_f=fi, out_f=fo,
                         nrows=nrows, nblk=fo // nrows, seg_off=seg_off,
                         seg_idx=len(segs)))
        seg_off += nnzp
    return segs, seg_off


_SEGS, _CAT_LEN = _seg_layout()
_NW = 32
_MAX_NNZP = max(s["nnzp"] for s in _SEGS)

_PB_ROWS = 320
_PBC = {"b1": (0, 8), "b2": (8, 16), "b3": (24, 32), "bm1": (56, 64),
        "bm2": (120, 3)}
_PBC_ROWS = 128
_PC64 = {"wrtr": (0, ENC), "wrtc": (ENC, R_SZ), "wrts": (ENC + R_SZ, C_SZ)}
_PC64_ROWS = ENC + R_SZ + C_SZ
_PCC_SEGS = [("bwr", R_SZ), ("brr", R_SZ), ("bwc", C_SZ), ("brc", C_SZ),
             ("bws", S_SZ), ("brs", S_SZ), ("bpr", R_SZ), ("bpc", C_SZ),
             ("bps", S_SZ), ("brtr", 64), ("brtc", 64), ("brts", 64),
             ("taur", R_SZ), ("tauc", C_SZ), ("taus", S_SZ),
             ("wfc", R_SZ), ("wg", HIDDEN), ("bfc", 1), ("bg", 1)]
_PCC = {}
_o = 0
for _nm, _n in _PCC_SEGS:
    _PCC[_nm] = (_o, _n)
    _o += _n
_PCC_ROWS = _o


def _prep_body(*args):
    ivs = args[:2 * len(_SEGS)]
    (k1_ref, b1_ref, k2_ref, b2_ref, k3_ref, b3_ref,
     wm1_ref, bm1_ref, wm2_ref, bm2_ref,
     wrtr_ref, wrtc_ref, wrts_ref,
     bwr_ref, brr_ref, bwc_ref, brc_ref, bws_ref, brs_ref,
     bpr_ref, bpc_ref, bps_ref, brtr_ref, brtc_ref, brts_ref,
     taur_ref, tauc_ref, taus_ref, wfc_ref, wg_ref, bfc_ref, bg_ref,
     bproj_ref) = args[2 * len(_SEGS):2 * len(_SEGS) + 33]
    (ci_ref, cj_ref, cv_ref, pb_ref, pbc_ref, pc64_ref, pcc_ref,
     pa_ref) = args[2 * len(_SEGS) + 33:]

    for s in _SEGS:
        k = s["seg_idx"]
        ij = ivs[2 * k][...]
        v = ivs[2 * k + 1][...]
        pad = s["nnzp"] - s["nnz"]
        off = s["seg_off"]
        ci_ref[pl.ds(off, s["nnzp"])] = jnp.pad(ij[0], (0, pad))
        cj_ref[pl.ds(off, s["nnzp"])] = jnp.pad(ij[1], (0, pad))
        cv_ref[pl.ds(off, s["nnzp"])] = jnp.pad(v, (0, pad))

    pb_ref[0:8, 0:27] = k1_ref[...].reshape(8, 27)
    pb_ref[8:24, 0:216] = k2_ref[...].reshape(16, 216)
    pb_ref[24:56, 0:432] = k3_ref[...].reshape(32, 432)
    pb_ref[56:316, 0:64] = wm1_ref[...]
    pb_ref[316:319, 0:64] = wm2_ref[...].T

    col = lambda x: x.reshape(-1, 1)
    for (o, n), ref in [(_PBC["b1"], b1_ref), (_PBC["b2"], b2_ref),
                        (_PBC["b3"], b3_ref), (_PBC["bm1"], bm1_ref),
                        (_PBC["bm2"], bm2_ref)]:
        pbc_ref[pl.ds(o, n), :] = col(ref[...])

    pc64_ref[pl.ds(*_PC64["wrtr"]), :] = wrtr_ref[...]
    pc64_ref[pl.ds(*_PC64["wrtc"]), :] = wrtc_ref[...]
    pc64_ref[pl.ds(*_PC64["wrts"]), :] = wrts_ref[...]

    for nm, ref in [("bwr", bwr_ref), ("brr", brr_ref), ("bwc", bwc_ref),
                    ("brc", brc_ref), ("bws", bws_ref), ("brs", brs_ref),
                    ("bpr", bpr_ref), ("bpc", bpc_ref), ("bps", bps_ref),
                    ("brtr", brtr_ref), ("brtc", brtc_ref),
                    ("brts", brts_ref), ("taur", taur_ref),
                    ("tauc", tauc_ref), ("taus", taus_ref),
                    ("bfc", bfc_ref), ("bg", bg_ref)]:
        o, n = _PCC[nm]
        pcc_ref[pl.ds(o, n), :] = col(ref[...])
    pcc_ref[pl.ds(*_PCC["wfc"]), :] = wfc_ref[...]
    pcc_ref[pl.ds(*_PCC["wg"]), :] = wg_ref[...]

    pa_ref[...] = col(bproj_ref[...])


def _prep(ins):
    full = lambda a: pl.BlockSpec(a.shape, lambda: tuple(0 for _ in a.shape))
    out_shapes = [
        jax.ShapeDtypeStruct((_CAT_LEN,), jnp.int32),
        jax.ShapeDtypeStruct((_CAT_LEN,), jnp.int32),
        jax.ShapeDtypeStruct((_CAT_LEN,), jnp.float32),
        jax.ShapeDtypeStruct((_PB_ROWS, 432), jnp.float32),
        jax.ShapeDtypeStruct((_PBC_ROWS, 1), jnp.float32),
        jax.ShapeDtypeStruct((_PC64_ROWS, 64), jnp.float32),
        jax.ShapeDtypeStruct((_PCC_ROWS, 1), jnp.float32),
        jax.ShapeDtypeStruct((INPUT_SIZE, 1), jnp.float32),
    ]
    return pl.pallas_call(
        _prep_body,
        in_specs=[full(a) for a in ins],
        out_specs=[pl.BlockSpec(o.shape, functools.partial(
                       lambda n: (0,) * n, len(o.shape)))
                   for o in out_shapes],
        out_shape=out_shapes,
        interpret=_INTERPRET,
    )(*ins)


def _proj_body(x_ref, wp_ref, bT_ref, out_ref):
    out_ref[...] = lax.dot_general(
        wp_ref[...], x_ref[...], (((0,), (1,)), ((), ())),
        preferred_element_type=jnp.float32) + bT_ref[...]


def _proj(x, W_proj, bcol):
    FB = 512
    grid = (INPUT_SIZE // FB,)
    return pl.pallas_call(
        _proj_body,
        grid=grid,
        in_specs=[
            pl.BlockSpec((B, INPUT_SIZE), lambda i: (0, 0)),
            pl.BlockSpec((INPUT_SIZE, FB), lambda i: (0, i)),
            pl.BlockSpec((FB, 1), lambda i: (i, 0)),
        ],
        out_specs=pl.BlockSpec((FB, B), lambda i: (i, 0)),
        out_shape=jax.ShapeDtypeStruct((BASE ** 3, B), jnp.float32),
        interpret=_INTERPRET,
    )(x, W_proj, bcol)


def _dec(v, axis, d):
    D = v.shape[axis]
    newshape = v.shape[:axis] + (D // 2, 2) + v.shape[axis + 1:]
    vr = v.reshape(newshape)
    ve = lax.index_in_dim(vr, 0, axis + 1, keepdims=False)
    vo = lax.index_in_dim(vr, 1, axis + 1, keepdims=False)
    if d == 1:
        return ve
    if d == 2:
        return vo
    pad = jnp.zeros_like(lax.slice_in_dim(vo, 0, 1, axis=axis))
    return lax.concatenate([pad, lax.slice_in_dim(vo, 0, D // 2 - 1, axis=axis)],
                           dimension=axis)


def _conv3d(v, kr, b):
    Cin, D = v.shape[0], v.shape[1]
    Bb = v.shape[-1]
    D2 = D // 2
    patches = []
    for d1 in range(3):
        u1 = _dec(v, 1, d1)
        for d2 in range(3):
            u2 = _dec(u1, 2, d2)
            for d3 in range(3):
                patches.append(_dec(u2, 3, d3))
    S = jnp.stack(patches, axis=1)
    S = S.reshape(Cin * 27, D2 * D2 * D2 * Bb)
    y = lax.dot_general(kr, S, (((1,), (0,)), ((), ())),
                        preferred_element_type=jnp.float32)
    y = jax.nn.relu(y.reshape(-1, D2 * D2 * D2, Bb) + b[:, :, None])
    return y.reshape(-1, D2, D2, D2, Bb)


def _enc_body(zT_ref, pb_ref, pbc_ref, e_ref, m_ref):
    Bb = zT_ref.shape[-1]
    bc = lambda nm: pbc_ref[pl.ds(*_PBC[nm]), :]
    v = zT_ref[...].reshape(1, BASE, BASE, BASE, Bb)
    y1 = _conv3d(v, pb_ref[0:8, 0:27], bc("b1"))
    y2 = _conv3d(y1, pb_ref[8:24, 0:216], bc("b2"))
    y3 = _conv3d(y2, pb_ref[24:56, 0:432], bc("b3"))
    e = y3.reshape(ENC, Bb)
    mean = jnp.mean(e, axis=0, keepdims=True)
    std = jnp.sqrt(jnp.mean((e - mean) ** 2, axis=0, keepdims=True))
    mx = jnp.max(e, axis=0, keepdims=True)
    mn = jnp.min(e, axis=0, keepdims=True)
    cat = jnp.concatenate([e, mean, std, mx, mn], axis=0)
    t1 = jnp.tanh(lax.dot_general(pb_ref[56:316, 0:64], cat,
                                  (((0,), (0,)), ((), ())),
                                  preferred_element_type=jnp.float32)
                  + bc("bm1"))
    lg = lax.dot_general(pb_ref[316:319, 0:64], t1, (((1,), (0,)), ((), ())),
                         preferred_element_type=jnp.float32) + bc("bm2")
    lg = lg - jnp.max(lg, axis=0, keepdims=True)
    ex = jnp.exp(lg)
    m_ref[...] = ex / jnp.sum(ex, axis=0, keepdims=True)
    e_ref[...] = e


def _encode(zT, pb, pbc):
    BB = 128
    grid = (B // BB,)
    return pl.pallas_call(
        _enc_body,
        grid=grid,
        in_specs=[
            pl.BlockSpec((BASE ** 3, BB), lambda i: (0, i)),
            pl.BlockSpec(pb.shape, lambda i: (0, 0)),
            pl.BlockSpec(pbc.shape, lambda i: (0, 0)),
        ],
        out_specs=[
            pl.BlockSpec((ENC, BB), lambda i: (0, i)),
            pl.BlockSpec((3, BB), lambda i: (0, i)),
        ],
        out_shape=[
            jax.ShapeDtypeStruct((ENC, B), jnp.float32),
            jax.ShapeDtypeStruct((3, B), jnp.float32),
        ],
        interpret=_INTERPRET,
    )(zT, pb, pbc)


def _mm(A, X):
    return lax.dot_general(A, X, (((1,), (0,)), ((), ())),
                           preferred_element_type=jnp.float32)


def _mm_phys(Wp, X, in_f):
    k = 1024 // in_f
    if k == 1:
        return _mm(Wp, X)
    outs = [_mm(Wp[:, i * in_f:(i + 1) * in_f], X) for i in range(k)]
    y = jnp.stack(outs, axis=1)
    return y.reshape(-1, X.shape[-1])


def _tm(X, W):
    return lax.dot_general(X, W, (((0,), (0,)), ((), ())),
                           preferred_element_type=jnp.float32)


def _rec_body(e_ref, mm_ref,
              wr_ref, rr_ref, wc_ref, rc_ref, ws_ref, rs_ref,
              pr_ref, pc_ref, ps_ref,
              pcc_ref, pc64_ref,
              wf_ref, bf_ref, wd_ref, bd_ref,
              out_ref):
    def _bp(nm):
        return pcc_ref[pl.ds(*_PCC[nm]), :]

    def _w64(nm):
        return pc64_ref[pl.ds(*_PC64[nm]), :]

    e = e_ref[...]
    mod = mm_ref[...]
    m0, m1, m2 = mod[0:1, :], mod[1:2, :], mod[2:3, :]

    a_r = 1.0 / (1.0 + _bp("taur"))
    a_c = 1.0 / (1.0 + _bp("tauc"))
    a_s = 1.0 / (1.0 + _bp("taus"))

    g_r = jax.nn.sigmoid(jnp.mean(_tm(_w64("wrtr"), e) + _bp("brtr"),
                                  axis=0, keepdims=True))
    wrp = _mm_phys(wr_ref[...], e, ENC) + _bp("bwr") + _bp("brr")
    h = a_r * jnp.tanh(wrp) * m0 * g_r
    pre = wrp + _mm_phys(rr_ref[...], h, R_SZ)
    h_r = (1.0 - a_r) * h + a_r * jnp.tanh(pre) * m0 * g_r

    g_c = jax.nn.sigmoid(jnp.mean(_tm(_w64("wrtc"), h_r) + _bp("brtc"),
                                  axis=0, keepdims=True))
    wcp = _mm_phys(wc_ref[...], h_r, R_SZ) + _bp("bwc") + _bp("brc")
    h = a_c * jnp.tanh(wcp) * m1 * g_c
    pre = wcp + _mm_phys(rc_ref[...], h, C_SZ)
    h_c = (1.0 - a_c) * h + a_c * jnp.tanh(pre) * m1 * g_c

    g_s = jax.nn.sigmoid(jnp.mean(_tm(_w64("wrts"), h_c) + _bp("brts"),
                                  axis=0, keepdims=True))
    wsp = _mm_phys(ws_ref[...], h_c, C_SZ) + _bp("bws") + _bp("brs")
    h = a_s * jnp.tanh(wsp) * m2 * g_s
    pre = wsp + _mm_phys(rs_ref[...], h, S_SZ)
    h_s = (1.0 - a_s) * h + a_s * jnp.tanh(pre) * m2 * g_s

    hr2 = h_r + jnp.tanh(_mm_phys(pr_ref[...], h_r, R_SZ) + _bp("bpr"))
    hc2 = h_c + jnp.tanh(_mm_phys(pc_ref[...], h_c, C_SZ) + _bp("bpc"))
    hs2 = h_s + jnp.tanh(_mm_phys(ps_ref[...], h_s, S_SZ) + _bp("bps"))
    hh = jnp.concatenate([hr2, hc2, hs2], axis=0)

    dec = _tm(hh, wd_ref[...]) + bd_ref[...]
    gate = jax.nn.sigmoid(_tm(hh, _bp("wg")) + _bp("bg"))
    flash = _tm(h_r, wf_ref[...]) + bf_ref[...]
    conf = jax.nn.sigmoid(_tm(h_r, _bp("wfc")) + _bp("bfc"))
    out_ref[...] = conf * flash + (1.0 - conf) * gate * dec


def _recurrent(e, mod, Ws, pcc, pc64, Wf, bf, Wd, bd):
    args = [e, mod] + Ws + [pcc, pc64, Wf, bf, Wd, bd]
    full = lambda a: pl.BlockSpec(a.shape, lambda: tuple(0 for _ in a.shape))
    return pl.pallas_call(
        _rec_body,
        in_specs=[full(a) for a in args],
        out_specs=pl.BlockSpec((B, OUTPUT), lambda: (0, 0)),
        out_shape=jax.ShapeDtypeStruct((B, OUTPUT), jnp.float32),
        interpret=_INTERPRET,
    )(*args)


def _sc_scatter_body(*args):
    ci_hbm, cj_hbm, cv_hbm = args[:3]
    outs = args[3:3 + len(_SEGS)]
    buf, ib, jb, vb, cidx, cval = args[3 + len(_SEGS):]
    wid = lax.axis_index("c") * 16 + lax.axis_index("s")
    lane = lax.broadcasted_iota(jnp.int32, (16,), 0)
    zeros16 = jnp.zeros((16,), jnp.float32)

    def zero_body(r, _):
        for k in range(64):
            buf[r, pl.ds(k * 16, 16)] = zeros16
        return 0

    lax.fori_loop(0, 64, zero_body, 0)

    tstart = 0
    for s in _SEGS:
        nnzp = s["nnzp"]
        nchunks = nnzp // 16
        in_f = s["in_f"]
        nrows = s["nrows"]
        out_hbm = outs[s["seg_idx"]]

        def blk_body(blk, _, s=s, tstart=tstart, nnzp=nnzp, nchunks=nchunks,
                     in_f=in_f, nrows=nrows, out_hbm=out_hbm):
            t = tstart + blk
            owner = lax.rem(t, _NW)

            @pl.when(owner == wid)
            def _():
                pltpu.sync_copy(ci_hbm.at[pl.ds(s["seg_off"], nnzp)],
                                ib.at[pl.ds(0, nnzp)])
                pltpu.sync_copy(cj_hbm.at[pl.ds(s["seg_off"], nnzp)],
                                jb.at[pl.ds(0, nnzp)])
                pltpu.sync_copy(cv_hbm.at[pl.ds(s["seg_off"], nnzp)],
                                vb.at[pl.ds(0, nnzp)])
                row0 = blk * nrows

                def chunk_body(c, cnt):
                    i16 = ib[pl.ds(c * 16, 16)]
                    j16 = jb[pl.ds(c * 16, 16)]
                    v16 = vb[pl.ds(c * 16, 16)]
                    owned = (j16 >= row0) & (j16 < row0 + nrows)
                    local = jnp.where(owned, (j16 - row0) * in_f + i16, 0)
                    plsc.store_compressed(cidx.at[pl.ds(cnt, 16)], local,
                                          mask=owned)
                    plsc.store_compressed(cval.at[pl.ds(cnt, 16)], v16,
                                          mask=owned)
                    return cnt + jnp.sum(owned.astype(jnp.int32))

                cnt = lax.fori_loop(0, nchunks, chunk_body, 0)

                def scat_body(c, _):
                    li = cidx[pl.ds(c * 16, 16)]
                    lv = cval[pl.ds(c * 16, 16)]
                    r16 = lax.shift_right_logical(li, 10)
                    c16 = lax.bitwise_and(li, 1023)
                    valid = lane < (cnt - c * 16)
                    for l in range(16):
                        plsc.addupdate_scatter(buf, (r16, c16), lv,
                                               mask=valid & (lane == l))
                    return 0

                nsc = lax.div(cnt + 15, 16)
                lax.fori_loop(0, nsc, scat_body, 0)

                pltpu.sync_copy(buf, out_hbm.at[pl.ds(blk * 64, 64)])

                def rz_body(c, _):
                    li = cidx[pl.ds(c * 16, 16)]
                    r16 = lax.shift_right_logical(li, 10)
                    c16 = lax.bitwise_and(li, 1023)
                    valid = lane < (cnt - c * 16)
                    plsc.store_scatter(buf, (r16, c16), zeros16, mask=valid)
                    return 0

                lax.fori_loop(0, nsc, rz_body, 0)

            return 0

        lax.fori_loop(0, s["nblk"], blk_body, 0)
        tstart += s["nblk"]


def _materialize_dense_T(ci, cj, cv):
    phys = pl.kernel(
        _sc_scatter_body,
        out_type=[jax.ShapeDtypeStruct((s["in_f"] * s["out_f"] // 1024, 1024),
                                       jnp.float32) for s in _SEGS],
        mesh=plsc.VectorSubcoreMesh(core_axis_name="c", subcore_axis_name="s"),
        compiler_params=pltpu.CompilerParams(needs_layout_passes=False,
                                             skip_device_barrier=True),
        scratch_types=[
            pltpu.VMEM((64, 1024), jnp.float32),
            pltpu.VMEM((_MAX_NNZP,), jnp.int32),
            pltpu.VMEM((_MAX_NNZP,), jnp.int32),
            pltpu.VMEM((_MAX_NNZP,), jnp.float32),
            pltpu.VMEM((_MAX_NNZP + 16,), jnp.int32),
            pltpu.VMEM((_MAX_NNZP + 16,), jnp.float32),
        ],
    )(ci, cj, cv)

    ws = {s["nm"]: f for s, f in zip(_SEGS, phys)}
    return [ws[nm] for nm, _, _ in _SPECS]


def kernel(x, W_proj, b_proj, k_conv1, b_conv1, k_conv2, b_conv2, k_conv3, b_conv3,
           idx_wr, val_wr, b_wr, idx_rr, val_rr, b_rr,
           idx_wc, val_wc, b_wc, idx_rc, val_rc, b_rc,
           idx_ws, val_ws, b_ws, idx_rs, val_rs, b_rs,
           idx_pr, val_pr, b_pr, idx_pc, val_pc, b_pc, idx_ps, val_ps, b_ps,
           Wm1, bm1, Wm2, bm2,
           Wrt_r, brt_r, Wrt_c, brt_c, Wrt_s, brt_s,
           tau_r, tau_c, tau_s,
           Wf, bf, Wfc, bfc, Wd, bd, Wg, bg):
    by_name = {"wr": (idx_wr, val_wr), "rr": (idx_rr, val_rr),
               "wc": (idx_wc, val_wc), "rc": (idx_rc, val_rc),
               "ws": (idx_ws, val_ws), "rs": (idx_rs, val_rs),
               "pr": (idx_pr, val_pr), "pc": (idx_pc, val_pc),
               "ps": (idx_ps, val_ps)}
    ins = []
    for s in _SEGS:
        idx, v = by_name[s["nm"]]
        ins += [idx, v]
    ins += [k_conv1, b_conv1, k_conv2, b_conv2, k_conv3, b_conv3,
            Wm1, bm1, Wm2, bm2,
            Wrt_r, Wrt_c, Wrt_s,
            b_wr, b_rr, b_wc, b_rc, b_ws, b_rs, b_pr, b_pc, b_ps,
            brt_r, brt_c, brt_s, tau_r, tau_c, tau_s,
            Wfc, Wg, bfc, bg, b_proj]
    ci, cj, cv, pb, pbc, pc64, pcc, pa = _prep(ins)

    Ws = _materialize_dense_T(ci, cj, cv)

    zT = _proj(x, W_proj, pa)
    e, mod = _encode(zT, pb, pbc)

    row = lambda b: b.reshape(1, -1)
    return _recurrent(e, mod, Ws, pcc, pc64, Wf, row(bf), Wd, row(bd))
